# baseline (speedup 1.0000x reference)
import functools
import math

import numpy as np
import jax
import jax.numpy as jnp
from jax import lax
from jax.experimental import pallas as pl
from jax.experimental.pallas import tpu as pltpu
from jax.experimental.pallas import tpu_sc as plsc

_IM = 256
_OS = 320
_W = 6
_PADG = _OS + _W
_BETA = math.pi * ((_W / 1.25 * (1.25 - 0.5)) ** 2 - 0.8) ** 0.5
_NPTS = 65536
_NC = 8
_NW = 32
_PW = _NPTS // _NW
_B = 64
_NCH = _PW // _B
_NTAP = _W * _W


def _axis_matrix():
    idx = np.arange(_IM, dtype=np.float64)
    ap = np.sqrt(_BETA ** 2 - (np.pi * _W * (idx - _IM // 2) / _OS) ** 2)
    ap = ap / np.sinh(ap)
    pb = (_OS - _IM) // 2
    u = np.arange(_OS, dtype=np.float64)
    m = np.exp(-2j * np.pi * np.outer(u - _OS // 2, idx + pb - _OS // 2) / _OS)
    m = m * ap[None, :] / 16.0
    return m


def _split(a):
    hi = a.astype(jnp.bfloat16).astype(jnp.float32)
    return hi, a - hi


def _dot3(a_hi, a_lo, b_hi, b_lo):
    d = functools.partial(jnp.dot, preferred_element_type=jnp.float32)
    return d(a_hi, b_hi) + (d(a_hi, b_lo) + d(a_lo, b_hi))


def _dense_body(img_ref, myr_ref, myi_ref, mxrt_ref, mxit_ref, gr_ref, gi_ref):
    x_hi, x_lo = _split(img_ref[0])
    myr_hi, myr_lo = _split(myr_ref[...])
    myi_hi, myi_lo = _split(myi_ref[...])
    mxrt_hi, mxrt_lo = _split(mxrt_ref[...])
    mxit_hi, mxit_lo = _split(mxit_ref[...])
    ar = _dot3(myr_hi, myr_lo, x_hi, x_lo)
    ai = _dot3(myi_hi, myi_lo, x_hi, x_lo)
    ar_hi, ar_lo = _split(ar)
    ai_hi, ai_lo = _split(ai)
    gr_ref[0] = (_dot3(ar_hi, ar_lo, mxrt_hi, mxrt_lo)
                 - _dot3(ai_hi, ai_lo, mxit_hi, mxit_lo))
    gi_ref[0] = (_dot3(ar_hi, ar_lo, mxit_hi, mxit_lo)
                 + _dot3(ai_hi, ai_lo, mxrt_hi, mxrt_lo))


def _kb(x):
    mask = jnp.abs(x) <= 1.0
    xx = _BETA * jnp.sqrt(jnp.clip(1.0 - x * x, 1e-12))
    t2 = (xx / 3.75) * (xx / 3.75)
    small = 1.0 + t2 * (3.5156229 + t2 * (3.0899424 + t2 * (1.2067492
            + t2 * (0.2659732 + t2 * (0.0360768 + t2 * 0.0045813)))))
    ti = 3.75 / jnp.maximum(xx, 1e-12)
    big = lax.rsqrt(jnp.maximum(xx, 1e-12)) * jnp.exp(xx) * (
        0.39894228 + ti * (0.01328592 + ti * (0.00225319 + ti * (-0.00157565
        + ti * (0.00916281 + ti * (-0.02057706 + ti * (0.02635537
        + ti * (-0.01647633 + ti * 0.00392377))))))))
    r = jnp.where(xx < 3.75, small, big)
    return jnp.where(mask, r, 0.0)


def _weights_body(ty_ref, tx_ref, wt_ref, base_ref):
    ky = ty_ref[...] * 1.25 + 160.0
    kx = tx_ref[...] * 1.25 + 160.0
    y0f = jnp.ceil(ky - 3.0)
    x0f = jnp.ceil(kx - 3.0)
    y0 = y0f.astype(jnp.int32)
    x0 = x0f.astype(jnp.int32)
    base_ref[...] = (y0 + 3) * _PADG + (x0 + 3)
    for dy in range(_W):
        wt_ref[dy] = _kb((y0f + dy - ky) / 3.0)
    for dx in range(_W):
        wt_ref[_W + dx] = _kb((x0f + dx - kx) / 3.0) * (1.0 / _NTAP)
    zero = jnp.zeros_like(ky)
    for k in range(2 * _W, 16):
        wt_ref[k] = zero


def _sc_interp(table, base, wtc):
    mesh = plsc.VectorSubcoreMesh(core_axis_name="c", subcore_axis_name="s",
                                  num_cores=2, num_subcores=16)

    @functools.partial(
        pl.kernel,
        out_type=jax.ShapeDtypeStruct((2 * _NC, _NPTS), jnp.float32),
        mesh=mesh,
        compiler_params=pltpu.CompilerParams(
            needs_layout_passes=False, use_tc_tiling_on_sc=False),
        scratch_types=[
            pltpu.VMEM((_PW,), jnp.int32),
            pltpu.VMEM((2, _NTAP, _B), jnp.int32),
            pltpu.VMEM((2, 2 * _NC, _B), jnp.float32),
            pltpu.VMEM((2, _NTAP * _B, 2 * _NC), jnp.float32),
            pltpu.VMEM((2, 2 * _NC, _B), jnp.float32),
            pltpu.SemaphoreType.DMA,
            pltpu.SemaphoreType.DMA,
            pltpu.SemaphoreType.DMA,
        ],
    )
    def kfn(table_hbm, base_hbm, wt_hbm, out_hbm,
            base_v, idx_v, wt_v, cells_v, out_v, sem_g, sem_w, sem_o):
        wid = lax.axis_index("s") * 2 + lax.axis_index("c")
        pstart = wid * _PW
        pltpu.sync_copy(base_hbm.at[pl.ds(pstart, _PW)], base_v)
        iota16 = lax.iota(jnp.int32, 16)

        def fire(ch):
            buf = lax.rem(ch, 2)
            cstart = pstart + ch * _B
            pltpu.make_async_copy(
                wt_hbm.at[:, pl.ds(cstart, _B)], wt_v.at[buf], sem_w).start()
            b16 = [base_v[pl.ds(ch * _B + v * 16, 16)]
                   for v in range(_B // 16)]

            @plsc.parallel_loop(0, _NTAP, unroll=2)
            def fire_tap(j):
                dy = j // _W
                off = dy * _PADG + (j - dy * _W)
                for v in range(_B // 16):
                    idx_v[buf, j, pl.ds(v * 16, 16)] = b16[v] + off
                pltpu.make_async_copy(
                    table_hbm.at[idx_v.at[buf, j]],
                    cells_v.at[buf, pl.ds(j * _B, _B)], sem_g).start()

        def drain(ch):
            buf = lax.rem(ch, 2)
            pltpu.make_async_copy(
                wt_hbm.at[:, pl.ds(pstart, _B)], wt_v.at[buf], sem_w).wait()

            def drain_tap(j, c):
                pltpu.make_async_copy(
                    table_hbm.at[idx_v.at[buf, j]],
                    cells_v.at[buf, pl.ds(j * _B, _B)], sem_g).wait()
                return c
            lax.fori_loop(0, _NTAP, drain_tap, 0)

        def out_wait():
            pltpu.make_async_copy(
                out_v.at[0], out_hbm.at[:, pl.ds(pstart, _B)], sem_o).wait()

        fire(0)

        def chunk_body(ch, carry):
            buf = lax.rem(ch, 2)
            cstart = pstart + ch * _B
            drain(ch)

            @pl.when(ch + 1 < _NCH)
            def _():
                fire(ch + 1)

            @pl.when(ch >= 2)
            def _():
                out_wait()

            bufsplat = jnp.full((16,), buf, jnp.int32)

            @plsc.parallel_loop(0, _B, unroll=2)
            def point_body(b):
                bsplat = jnp.full((16,), b, jnp.int32)
                wys = [plsc.load_gather(
                    wt_v, [bufsplat, jnp.full((16,), dy, jnp.int32), bsplat])
                    for dy in range(_W)]
                wxs = [plsc.load_gather(
                    wt_v, [bufsplat, jnp.full((16,), _W + dx, jnp.int32),
                           bsplat])
                    for dx in range(_W)]
                w = [[wys[dy] * wxs[dx] for dx in range(_W)]
                     for dy in range(_W)]
                accs = []
                for dy in range(_W):
                    a = w[dy][0] * cells_v[buf, (dy * _W) * _B + b]
                    for dx in range(1, _W):
                        a = a + w[dy][dx] * cells_v[buf, (dy * _W + dx) * _B + b]
                    accs.append(a)
                acc = (((accs[0] + accs[1]) + (accs[2] + accs[3]))
                       + (accs[4] + accs[5]))
                plsc.store_scatter(out_v, [bufsplat, iota16, bsplat], acc)
            pltpu.make_async_copy(
                out_v.at[buf], out_hbm.at[:, pl.ds(cstart, _B)], sem_o).start()
            return carry
        lax.fori_loop(0, _NCH, chunk_body, 0)
        out_wait()
        out_wait()

    return kfn(table, base, wtc)


def kernel(img, trj):
    img2 = img[0]
    m = _axis_matrix()
    myr = jnp.asarray(m.real, jnp.float32)
    myi = jnp.asarray(m.imag, jnp.float32)
    mxrt = jnp.asarray(m.real.T, jnp.float32)
    mxit = jnp.asarray(m.imag.T, jnp.float32)

    gr, gi = pl.pallas_call(
        _dense_body,
        grid=(_NC,),
        in_specs=[
            pl.BlockSpec((1, _IM, _IM), lambda i: (i, 0, 0)),
            pl.BlockSpec((_OS, _IM), lambda i: (0, 0)),
            pl.BlockSpec((_OS, _IM), lambda i: (0, 0)),
            pl.BlockSpec((_IM, _OS), lambda i: (0, 0)),
            pl.BlockSpec((_IM, _OS), lambda i: (0, 0)),
        ],
        out_specs=[
            pl.BlockSpec((1, _OS, _OS), lambda i: (i, 0, 0)),
            pl.BlockSpec((1, _OS, _OS), lambda i: (i, 0, 0)),
        ],
        out_shape=[
            jax.ShapeDtypeStruct((_NC, _OS, _OS), jnp.float32),
            jax.ShapeDtypeStruct((_NC, _OS, _OS), jnp.float32),
        ],
    )(img2, myr, myi, mxrt, mxit)

    tab = jnp.concatenate([gr, gi], axis=0).transpose(1, 2, 0)
    tab = jnp.pad(tab, ((3, 3), (3, 3), (0, 0)), mode="wrap")
    tab = tab.reshape(_PADG * _PADG, 2 * _NC)

    trjf = trj.reshape(_NPTS, 2)
    ty = trjf[:, 0].reshape(512, 128)
    tx = trjf[:, 1].reshape(512, 128)
    wt, base = pl.pallas_call(
        _weights_body,
        out_shape=[
            jax.ShapeDtypeStruct((16, 512, 128), jnp.float32),
            jax.ShapeDtypeStruct((512, 128), jnp.int32),
        ],
    )(ty, tx)
    wcm = wt.reshape(16, _NPTS)
    basef = base.reshape(_NPTS)

    o = _sc_interp(tab, basef, wcm)
    out = (o[:_NC] + 1j * o[_NC:]).astype(jnp.complex64)
    return out.reshape(1, _NC, 16, 4096)


# speedup vs baseline: 1.0586x; 1.0586x over previous
import functools
import math

import numpy as np
import jax
import jax.numpy as jnp
from jax import lax
from jax.experimental import pallas as pl
from jax.experimental.pallas import tpu as pltpu
from jax.experimental.pallas import tpu_sc as plsc

_IM = 256
_OS = 320
_W = 6
_PADG = _OS + _W
_BETA = math.pi * ((_W / 1.25 * (1.25 - 0.5)) ** 2 - 0.8) ** 0.5
_NPTS = 65536
_NC = 8
_NW = 32
_PW = _NPTS // _NW
_B = 64
_NCH = _PW // _B
_NTAP = _W * _W


def _axis_matrix():
    idx = np.arange(_IM, dtype=np.float64)
    ap = np.sqrt(_BETA ** 2 - (np.pi * _W * (idx - _IM // 2) / _OS) ** 2)
    ap = ap / np.sinh(ap)
    pb = (_OS - _IM) // 2
    u = np.arange(_OS, dtype=np.float64)
    m = np.exp(-2j * np.pi * np.outer(u - _OS // 2, idx + pb - _OS // 2) / _OS)
    m = m * ap[None, :] / 16.0
    return m


def _split(a):
    hi = a.astype(jnp.bfloat16).astype(jnp.float32)
    return hi, a - hi


def _dot3(a_hi, a_lo, b_hi, b_lo):
    d = functools.partial(jnp.dot, preferred_element_type=jnp.float32)
    return d(a_hi, b_hi) + (d(a_hi, b_lo) + d(a_lo, b_hi))


def _dense_body(img_ref, myr_ref, myi_ref, mxrt_ref, mxit_ref, gr_ref, gi_ref):
    x_hi, x_lo = _split(img_ref[0])
    myr_hi, myr_lo = _split(myr_ref[...])
    myi_hi, myi_lo = _split(myi_ref[...])
    mxrt_hi, mxrt_lo = _split(mxrt_ref[...])
    mxit_hi, mxit_lo = _split(mxit_ref[...])
    ar = _dot3(myr_hi, myr_lo, x_hi, x_lo)
    ai = _dot3(myi_hi, myi_lo, x_hi, x_lo)
    ar_hi, ar_lo = _split(ar)
    ai_hi, ai_lo = _split(ai)
    gr_ref[0] = (_dot3(ar_hi, ar_lo, mxrt_hi, mxrt_lo)
                 - _dot3(ai_hi, ai_lo, mxit_hi, mxit_lo))
    gi_ref[0] = (_dot3(ar_hi, ar_lo, mxit_hi, mxit_lo)
                 + _dot3(ai_hi, ai_lo, mxrt_hi, mxrt_lo))


def _kb(x):
    mask = jnp.abs(x) <= 1.0
    xx = _BETA * jnp.sqrt(jnp.clip(1.0 - x * x, 1e-12))
    t2 = (xx / 3.75) * (xx / 3.75)
    small = 1.0 + t2 * (3.5156229 + t2 * (3.0899424 + t2 * (1.2067492
            + t2 * (0.2659732 + t2 * (0.0360768 + t2 * 0.0045813)))))
    ti = 3.75 / jnp.maximum(xx, 1e-12)
    big = lax.rsqrt(jnp.maximum(xx, 1e-12)) * jnp.exp(xx) * (
        0.39894228 + ti * (0.01328592 + ti * (0.00225319 + ti * (-0.00157565
        + ti * (0.00916281 + ti * (-0.02057706 + ti * (0.02635537
        + ti * (-0.01647633 + ti * 0.00392377))))))))
    r = jnp.where(xx < 3.75, small, big)
    return jnp.where(mask, r, 0.0)


def _weights_body(ty_ref, tx_ref, wt_ref, base_ref):
    ky = ty_ref[...] * 1.25 + 160.0
    kx = tx_ref[...] * 1.25 + 160.0
    y0f = jnp.ceil(ky - 3.0)
    x0f = jnp.ceil(kx - 3.0)
    y0 = y0f.astype(jnp.int32)
    x0 = x0f.astype(jnp.int32)
    base_ref[...] = (y0 + 3) * _PADG + (x0 + 3)
    for dy in range(_W):
        wt_ref[dy] = _kb((y0f + dy - ky) / 3.0)
    for dx in range(_W):
        wt_ref[_W + dx] = _kb((x0f + dx - kx) / 3.0) * (1.0 / _NTAP)
    zero = jnp.zeros_like(ky)
    for k in range(2 * _W, 16):
        wt_ref[k] = zero


def _sc_interp(table, base, wtc):
    mesh = plsc.VectorSubcoreMesh(core_axis_name="c", subcore_axis_name="s",
                                  num_cores=2, num_subcores=16)

    @functools.partial(
        pl.kernel,
        out_type=jax.ShapeDtypeStruct((2 * _NC, _NPTS), jnp.float32),
        mesh=mesh,
        compiler_params=pltpu.CompilerParams(
            needs_layout_passes=False, use_tc_tiling_on_sc=False),
        scratch_types=[
            pltpu.VMEM((_PW,), jnp.int32),
            pltpu.VMEM((2, _NTAP, _B), jnp.int32),
            pltpu.VMEM((2, 2 * _NC, _B), jnp.float32),
            pltpu.VMEM((2, _NTAP * _B, 2 * _NC), jnp.float32),
            pltpu.VMEM((2, 2 * _NC, _B), jnp.float32),
            pltpu.SemaphoreType.DMA,
            pltpu.SemaphoreType.DMA,
            pltpu.SemaphoreType.DMA,
        ],
    )
    def kfn(table_hbm, base_hbm, wt_hbm, out_hbm,
            base_v, idx_v, wt_v, cells_v, out_v, sem_g, sem_w, sem_o):
        wid = lax.axis_index("s") * 2 + lax.axis_index("c")
        pstart = wid * _PW
        pltpu.sync_copy(base_hbm.at[pl.ds(pstart, _PW)], base_v)
        iota16 = lax.iota(jnp.int32, 16)

        def fire(ch):
            buf = lax.rem(ch, 2)
            cstart = pstart + ch * _B
            pltpu.make_async_copy(
                wt_hbm.at[:, pl.ds(cstart, _B)], wt_v.at[buf], sem_w).start()
            b16 = [base_v[pl.ds(ch * _B + v * 16, 16)]
                   for v in range(_B // 16)]

            @plsc.parallel_loop(0, _NTAP, unroll=2)
            def fire_tap(j):
                dy = j // _W
                off = dy * _PADG + (j - dy * _W)
                for v in range(_B // 16):
                    idx_v[buf, j, pl.ds(v * 16, 16)] = b16[v] + off
                pltpu.make_async_copy(
                    table_hbm.at[idx_v.at[buf, j]],
                    cells_v.at[buf, pl.ds(j * _B, _B)], sem_g).start()

        def drain(ch):
            buf = lax.rem(ch, 2)
            pltpu.make_async_copy(
                wt_hbm.at[:, pl.ds(pstart, _B)], wt_v.at[buf], sem_w).wait()

            def drain_tap(j, c):
                pltpu.make_async_copy(
                    table_hbm.at[idx_v.at[buf, j]],
                    cells_v.at[buf, pl.ds(j * _B, _B)], sem_g).wait()
                return c
            lax.fori_loop(0, _NTAP, drain_tap, 0)

        def out_wait():
            pltpu.make_async_copy(
                out_v.at[0], out_hbm.at[:, pl.ds(pstart, _B)], sem_o).wait()

        fire(0)

        def chunk_body(ch, carry):
            buf = lax.rem(ch, 2)
            cstart = pstart + ch * _B
            drain(ch)

            @pl.when(ch + 1 < _NCH)
            def _():
                fire(ch + 1)

            @pl.when(ch >= 2)
            def _():
                out_wait()

            bufsplat = jnp.full((16,), buf, jnp.int32)

            @plsc.parallel_loop(0, _B, unroll=1)
            def point_body(b):
                bsplat = jnp.full((16,), b, jnp.int32)
                wys = [plsc.load_gather(
                    wt_v, [bufsplat, jnp.full((16,), dy, jnp.int32), bsplat])
                    for dy in range(_W)]
                wxs = [plsc.load_gather(
                    wt_v, [bufsplat, jnp.full((16,), _W + dx, jnp.int32),
                           bsplat])
                    for dx in range(_W)]
                w = [[wys[dy] * wxs[dx] for dx in range(_W)]
                     for dy in range(_W)]
                accs = []
                for dy in range(_W):
                    a = w[dy][0] * cells_v[buf, (dy * _W) * _B + b]
                    for dx in range(1, _W):
                        a = a + w[dy][dx] * cells_v[buf, (dy * _W + dx) * _B + b]
                    accs.append(a)
                acc = (((accs[0] + accs[1]) + (accs[2] + accs[3]))
                       + (accs[4] + accs[5]))
                plsc.store_scatter(out_v, [bufsplat, iota16, bsplat], acc)
            pltpu.make_async_copy(
                out_v.at[buf], out_hbm.at[:, pl.ds(cstart, _B)], sem_o).start()
            return carry
        lax.fori_loop(0, _NCH, chunk_body, 0)
        out_wait()
        out_wait()

    return kfn(table, base, wtc)


def kernel(img, trj):
    img2 = img[0]
    m = _axis_matrix()
    myr = jnp.asarray(m.real, jnp.float32)
    myi = jnp.asarray(m.imag, jnp.float32)
    mxrt = jnp.asarray(m.real.T, jnp.float32)
    mxit = jnp.asarray(m.imag.T, jnp.float32)

    gr, gi = pl.pallas_call(
        _dense_body,
        grid=(_NC,),
        in_specs=[
            pl.BlockSpec((1, _IM, _IM), lambda i: (i, 0, 0)),
            pl.BlockSpec((_OS, _IM), lambda i: (0, 0)),
            pl.BlockSpec((_OS, _IM), lambda i: (0, 0)),
            pl.BlockSpec((_IM, _OS), lambda i: (0, 0)),
            pl.BlockSpec((_IM, _OS), lambda i: (0, 0)),
        ],
        out_specs=[
            pl.BlockSpec((1, _OS, _OS), lambda i: (i, 0, 0)),
            pl.BlockSpec((1, _OS, _OS), lambda i: (i, 0, 0)),
        ],
        out_shape=[
            jax.ShapeDtypeStruct((_NC, _OS, _OS), jnp.float32),
            jax.ShapeDtypeStruct((_NC, _OS, _OS), jnp.float32),
        ],
    )(img2, myr, myi, mxrt, mxit)

    tab = jnp.concatenate([gr, gi], axis=0).transpose(1, 2, 0)
    tab = jnp.pad(tab, ((3, 3), (3, 3), (0, 0)), mode="wrap")
    tab = tab.reshape(_PADG * _PADG, 2 * _NC)

    trjf = trj.reshape(_NPTS, 2)
    ty = trjf[:, 0].reshape(512, 128)
    tx = trjf[:, 1].reshape(512, 128)
    wt, base = pl.pallas_call(
        _weights_body,
        out_shape=[
            jax.ShapeDtypeStruct((16, 512, 128), jnp.float32),
            jax.ShapeDtypeStruct((512, 128), jnp.int32),
        ],
    )(ty, tx)
    wcm = wt.reshape(16, _NPTS)
    basef = base.reshape(_NPTS)

    o = _sc_interp(tab, basef, wcm)
    out = (o[:_NC] + 1j * o[_NC:]).astype(jnp.complex64)
    return out.reshape(1, _NC, 16, 4096)
